# Initial kernel scaffold; baseline (speedup 1.0000x reference)
import functools

import jax
import jax.numpy as jnp
from jax import lax
from jax.experimental import pallas as pl
from jax.experimental.pallas import tpu as pltpu
from jax.experimental.pallas import tpu_sc as plsc

B = 8
N = 16384
K = 64
M = 64
S = 32
KM_NITER = 10

NC, NS, L = 2, 16, 16
NW = NC * NS
COLS = B * K
CPW = COLS // NW
CAP = 256
REBUILD_AT = CAP - 64
FINF = jnp.float32(jnp.inf)
IMAX = jnp.int32(2**31 - 1)


def _kmeans_body(x_ref, out_ref):
    x = x_ref[0]

    rows = lax.broadcasted_iota(jnp.int32, (N, K), 0)
    cols = lax.broadcasted_iota(jnp.int32, (N, K), 1)
    e0 = (rows == cols).astype(jnp.float32)
    ct0 = lax.dot_general(x, e0, (((0,), (0,)), ((), ())),
                          preferred_element_type=jnp.float32)

    def it(_, ct):
        d2 = (x[:, 0:1] - ct[0:1, :]) ** 2 + (x[:, 1:2] - ct[1:2, :]) ** 2
        d2 = d2 + (x[:, 2:3] - ct[2:3, :]) ** 2
        minv = jnp.min(d2, axis=1, keepdims=True)
        eq = (d2 == minv)
        first = jnp.cumsum(eq.astype(jnp.int32), axis=1) == 1
        oh = (eq & first).astype(jnp.float32)
        sums = lax.dot_general(x, oh, (((0,), (0,)), ((), ())),
                               preferred_element_type=jnp.float32)
        cnt = jnp.sum(oh, axis=0, keepdims=True)
        return sums / cnt

    out_ref[0] = lax.fori_loop(0, KM_NITER, it, ct0)


def _kmeans(x):
    return pl.pallas_call(
        _kmeans_body,
        grid=(B,),
        in_specs=[pl.BlockSpec((1, N, 3), lambda b: (b, 0, 0))],
        out_specs=pl.BlockSpec((1, 3, K), lambda b: (b, 0, 0)),
        out_shape=jax.ShapeDtypeStruct((B, 3, K), jnp.float32),
    )(x)


def _sc_body(xt_hbm, ct_hbm, f0_hbm, out_hbm,
             x0, x1, x2, cb0, cb1, cb2, f0v,
             cand_v, cand_i, tv, ti, nb0, nb1, nb2, fdist, seli, outrow,
             cnt_s, tau_s):
    wid = lax.axis_index("s") * NC + lax.axis_index("c")
    b = wid // (K // CPW)

    pltpu.sync_copy(xt_hbm.at[b, 0], x0)
    pltpu.sync_copy(xt_hbm.at[b, 1], x1)
    pltpu.sync_copy(xt_hbm.at[b, 2], x2)
    pltpu.sync_copy(ct_hbm.at[0, pl.ds(wid * CPW, CPW)], cb0)
    pltpu.sync_copy(ct_hbm.at[1, pl.ds(wid * CPW, CPW)], cb1)
    pltpu.sync_copy(ct_hbm.at[2, pl.ds(wid * CPW, CPW)], cb2)
    pltpu.sync_copy(f0_hbm.at[pl.ds(wid * CPW, CPW)], f0v)

    lanes = lax.iota(jnp.int32, L)

    def rebuild():
        cnt = cnt_s[0]

        def pad(jj, _):
            v = cand_v[pl.ds(jj * L, L)]
            pos = lanes + jj * L
            cand_v[pl.ds(jj * L, L)] = jnp.where(pos >= cnt, FINF, v)
            return 0

        lax.fori_loop(0, CAP // L, pad, 0)

        def rnd(r, _):
            def scan1(jj, carry):
                bv, bi = carry
                v = cand_v[pl.ds(jj * L, L)]
                ii = cand_i[pl.ds(jj * L, L)]
                upd = (v < bv) | ((v == bv) & (ii < bi))
                return jnp.where(upd, v, bv), jnp.where(upd, ii, bi)

            bv, bi = lax.fori_loop(0, CAP // L, scan1,
                                   (jnp.full((L,), FINF),
                                    jnp.full((L,), IMAX)))
            mv = jnp.min(bv)
            im = jnp.min(jnp.where(bv == mv, bi, IMAX))

            def clear(jj, _):
                v = cand_v[pl.ds(jj * L, L)]
                ii = cand_i[pl.ds(jj * L, L)]
                hit = (v == mv) & (ii == im)
                cand_v[pl.ds(jj * L, L)] = jnp.where(hit, FINF, v)
                return 0

            lax.fori_loop(0, CAP // L, clear, 0)
            tv[r] = mv
            ti[r] = im
            return 0

        lax.fori_loop(0, M, rnd, 0)

        def writeback(jj, _):
            cand_v[pl.ds(jj * L, L)] = tv[pl.ds(jj * L, L)]
            cand_i[pl.ds(jj * L, L)] = ti[pl.ds(jj * L, L)]
            return 0

        lax.fori_loop(0, M // L, writeback, 0)

        def fill_inf(jj, _):
            cand_v[pl.ds(M + jj * L, L)] = jnp.full((L,), FINF)
            return 0

        lax.fori_loop(0, (CAP - M) // L, fill_inf, 0)
        cnt_s[0] = M
        tau_s[0] = tv[M - 1]

    def column(cc, _):
        col = wid * CPW + cc
        c0 = cb0[cc]
        c1 = cb1[cc]
        c2 = cb2[cc]

        def fill(jj, _):
            cand_v[pl.ds(jj * L, L)] = jnp.full((L,), FINF)
            return 0

        lax.fori_loop(0, CAP // L, fill, 0)
        cnt_s[0] = 0
        tau_s[0] = FINF

        UNROLL = 4

        def step(j, _):
            @pl.when(cnt_s[0] > REBUILD_AT)
            def _():
                rebuild()

            tau = tau_s[0]
            base = j * (L * UNROLL)
            ds_ = []
            ms_ = []
            ns_ = []
            for u in range(UNROLL):
                off = base + u * L
                xv0 = x0[pl.ds(off, L)]
                xv1 = x1[pl.ds(off, L)]
                xv2 = x2[pl.ds(off, L)]
                d = (xv0 - c0) ** 2 + (xv1 - c1) ** 2
                d = d + (xv2 - c2) ** 2
                m = d < tau
                ds_.append(d)
                ms_.append(m)
                ns_.append(jnp.max(plsc.all_reduce_population_count(m)))
            total = ns_[0] + ns_[1] + ns_[2] + ns_[3]

            @pl.when(total > 0)
            def _():
                cnt = cnt_s[0]
                for u in range(UNROLL):
                    idxv = lanes + (base + u * L)
                    plsc.store_compressed(cand_v.at[pl.ds(cnt, L)],
                                          ds_[u], mask=ms_[u])
                    plsc.store_compressed(cand_i.at[pl.ds(cnt, L)],
                                          idxv, mask=ms_[u])
                    cnt = cnt + ns_[u]
                cnt_s[0] = cnt

            return 0

        lax.fori_loop(0, N // (L * UNROLL), step, 0)
        rebuild()

        for u in range(M // L):
            iv = cand_i[pl.ds(u * L, L)]
            nb0[pl.ds(u * L, L)] = plsc.load_gather(x0, [iv])
            nb1[pl.ds(u * L, L)] = plsc.load_gather(x1, [iv])
            nb2[pl.ds(u * L, L)] = plsc.load_gather(x2, [iv])
            fdist[pl.ds(u * L, L)] = jnp.full((L,), jnp.float32(1e10))

        def fps(it, far):
            seli[it] = far
            p0 = nb0[far]
            p1 = nb1[far]
            p2 = nb2[far]
            bv = jnp.full((L,), jnp.float32(-1.0))
            bp = jnp.full((L,), jnp.int32(M))
            for u in range(M // L):
                nv0 = nb0[pl.ds(u * L, L)]
                nv1 = nb1[pl.ds(u * L, L)]
                nv2 = nb2[pl.ds(u * L, L)]
                dd = (nv0 - p0) ** 2 + (nv1 - p1) ** 2
                dd = dd + (nv2 - p2) ** 2
                nd = jnp.minimum(fdist[pl.ds(u * L, L)], dd)
                fdist[pl.ds(u * L, L)] = nd
                upd = nd > bv
                bv = jnp.where(upd, nd, bv)
                bp = jnp.where(upd, lanes + u * L, bp)
            mx = jnp.max(bv)
            return jnp.min(jnp.where(bv == mx, bp, IMAX))

        lax.fori_loop(0, S, fps, f0v[cc])

        for u in range(S // L):
            iv = seli[pl.ds(u * L, L)]
            g0 = plsc.load_gather(nb0, [iv])
            g1 = plsc.load_gather(nb1, [iv])
            g2 = plsc.load_gather(nb2, [iv])
            pos = (lanes + u * L) * 3
            plsc.store_scatter(outrow, [pos], g0)
            plsc.store_scatter(outrow, [pos + 1], g1)
            plsc.store_scatter(outrow, [pos + 2], g2)

        pltpu.sync_copy(outrow, out_hbm.at[col])
        return 0

    lax.fori_loop(0, CPW, column, 0)


def _sc_select_fps(xt, ct512, f0):
    mesh = plsc.VectorSubcoreMesh(core_axis_name="c", subcore_axis_name="s")
    fn = pl.kernel(
        _sc_body,
        out_type=jax.ShapeDtypeStruct((COLS, S * 3), jnp.float32),
        mesh=mesh,
        scratch_types=[
            pltpu.VMEM((N,), jnp.float32),
            pltpu.VMEM((N,), jnp.float32),
            pltpu.VMEM((N,), jnp.float32),
            pltpu.VMEM((CPW,), jnp.float32),
            pltpu.VMEM((CPW,), jnp.float32),
            pltpu.VMEM((CPW,), jnp.float32),
            pltpu.VMEM((CPW,), jnp.int32),
            pltpu.VMEM((CAP,), jnp.float32),
            pltpu.VMEM((CAP,), jnp.int32),
            pltpu.VMEM((M,), jnp.float32),
            pltpu.VMEM((M,), jnp.int32),
            pltpu.VMEM((M,), jnp.float32),
            pltpu.VMEM((M,), jnp.float32),
            pltpu.VMEM((M,), jnp.float32),
            pltpu.VMEM((M,), jnp.float32),
            pltpu.VMEM((S,), jnp.int32),
            pltpu.VMEM((S * 3,), jnp.float32),
            pltpu.SMEM((1,), jnp.int32),
            pltpu.SMEM((1,), jnp.float32),
        ],
    )
    return fn(xt, ct512, f0)


@jax.jit
def kernel(x):
    ct_all = _kmeans(x)
    centers = jnp.transpose(ct_all, (0, 2, 1))
    ct512 = jnp.transpose(ct_all, (1, 0, 2)).reshape(3, COLS)
    xt = jnp.transpose(x, (0, 2, 1))
    f0 = jax.random.randint(jax.random.key(1), (B, K), 0, M,
                            dtype=jnp.int32).reshape(COLS)
    out = _sc_select_fps(xt, ct512, f0)
    new_xyz = out.reshape(B, K * S, 3)
    return new_xyz, centers


# speedup vs baseline: 1.4557x; 1.4557x over previous
import jax
import jax.numpy as jnp
import numpy as np
from jax import lax
from jax.experimental import pallas as pl
from jax.experimental.pallas import tpu as pltpu
from jax.experimental.pallas import tpu_sc as plsc

B = 8
N = 16384
K = 64
M = 64
S = 32
KM_NITER = 10

NC, NS, L = 2, 16, 16
NW = NC * NS
COLS = B * K
CPW = COLS // NW
CAP = 256
REBUILD_AT = CAP - 64
FINF = np.float32(np.inf)
IMAX = np.int32(2**31 - 1)


def _assign_body(x_ref, ct_ref, out_ref):
    x = x_ref[0]
    ct = ct_ref[0]
    d0 = x[:, 0:1] - ct[0:1, :]
    d1 = x[:, 1:2] - ct[1:2, :]
    d2c = x[:, 2:3] - ct[2:3, :]
    d = d0 * d0 + d1 * d1
    d = d + d2c * d2c
    out_ref[0] = jnp.argmin(d, axis=1).astype(jnp.int32)[:, None]


def _assign(x, ct):
    return pl.pallas_call(
        _assign_body,
        grid=(B,),
        in_specs=[
            pl.BlockSpec((1, N, 3), lambda b: (b, 0, 0)),
            pl.BlockSpec((1, 3, K), lambda b: (b, 0, 0)),
        ],
        out_specs=pl.BlockSpec((1, N, 1), lambda b: (b, 0, 0)),
        out_shape=jax.ShapeDtypeStruct((B, N, 1), jnp.int32),
    )(x, ct)


def _kmeans(x):
    c = x[:, :K, :]
    for _ in range(KM_NITER):
        ct = jnp.transpose(c, (0, 2, 1))
        cl = _assign(x, ct)[..., 0]
        c = jax.vmap(
            lambda xi, cli: jnp.zeros((K, 3), x.dtype).at[cli].add(xi))(x, cl)
        Ncl = jax.vmap(
            lambda cli: jnp.zeros((K,), x.dtype).at[cli].add(1.0))(cl)
        c = c / Ncl[:, :, None]
    return c


def _sc_body(xt_hbm, ct_hbm, f0_hbm, out_hbm,
             x0, x1, x2, cb0, cb1, cb2, f0v,
             cand_v, cand_i, tv, ti, nb0, nb1, nb2, fdist, seli, outrow,
             cnt_s, tau_s):
    wid = lax.axis_index("s") * NC + lax.axis_index("c")
    b = wid // (K // CPW)

    pltpu.sync_copy(xt_hbm.at[pl.ds((b * 3 + 0) * N, N)], x0)
    pltpu.sync_copy(xt_hbm.at[pl.ds((b * 3 + 1) * N, N)], x1)
    pltpu.sync_copy(xt_hbm.at[pl.ds((b * 3 + 2) * N, N)], x2)
    pltpu.sync_copy(ct_hbm.at[pl.ds(0 * COLS + wid * CPW, CPW)],
                    cb0.at[pl.ds(0, CPW)])
    pltpu.sync_copy(ct_hbm.at[pl.ds(1 * COLS + wid * CPW, CPW)],
                    cb1.at[pl.ds(0, CPW)])
    pltpu.sync_copy(ct_hbm.at[pl.ds(2 * COLS + wid * CPW, CPW)],
                    cb2.at[pl.ds(0, CPW)])
    pltpu.sync_copy(f0_hbm.at[pl.ds(wid * CPW, CPW)], f0v.at[pl.ds(0, CPW)])

    lanes = lax.iota(jnp.int32, L)
    lane0 = lanes == 0

    def vload1(ref, idx):
        return ref[pl.ds(idx, L)][0]

    def vstore1(ref, idx, val):
        plsc.store_scatter(ref, [jnp.full((L,), idx, jnp.int32)],
                           jnp.full((L,), val), mask=lane0)

    def rebuild():
        def rnd(r, _):
            def scan1(jj, carry):
                bv, bi = carry
                v = cand_v[pl.ds(jj * L, L)]
                ii = cand_i[pl.ds(jj * L, L)]
                upd = (v < bv) | ((v == bv) & (ii < bi))
                return jnp.where(upd, v, bv), jnp.where(upd, ii, bi)

            bv, bi = lax.fori_loop(0, CAP // L, scan1,
                                   (jnp.full((L,), FINF),
                                    jnp.full((L,), IMAX)))
            mv = jnp.min(bv)
            im = jnp.min(jnp.where(bv == mv, bi, IMAX))

            def clear(jj, _):
                v = cand_v[pl.ds(jj * L, L)]
                ii = cand_i[pl.ds(jj * L, L)]
                hit = (v == mv) & (ii == im)
                cand_v[pl.ds(jj * L, L)] = jnp.where(hit, FINF, v)
                return 0

            lax.fori_loop(0, CAP // L, clear, 0)
            vstore1(tv, r, mv)
            vstore1(ti, r, im)
            tau_s[0] = mv
            return 0

        lax.fori_loop(0, M, rnd, 0)

        def writeback(jj, _):
            cand_v[pl.ds(jj * L, L)] = tv[pl.ds(jj * L, L)]
            cand_i[pl.ds(jj * L, L)] = ti[pl.ds(jj * L, L)]
            return 0

        lax.fori_loop(0, M // L, writeback, 0)

        def fill_inf(jj, _):
            cand_v[pl.ds(M + jj * L, L)] = jnp.full((L,), FINF)
            return 0

        lax.fori_loop(0, (CAP - M) // L, fill_inf, 0)
        cnt_s[0] = M

    def column(cc, _):
        col = wid * CPW + cc
        c0 = vload1(cb0, cc)
        c1 = vload1(cb1, cc)
        c2 = vload1(cb2, cc)

        def fill(jj, _):
            cand_v[pl.ds(jj * L, L)] = jnp.full((L,), FINF)
            return 0

        lax.fori_loop(0, CAP // L, fill, 0)
        cnt_s[0] = 0
        tau_s[0] = FINF

        UNROLL = 4

        def step(j, _):
            @pl.when(cnt_s[0] > REBUILD_AT)
            def _():
                rebuild()

            tau = tau_s[0]
            base = j * (L * UNROLL)
            ds_ = []
            ms_ = []
            ns_ = []
            for u in range(UNROLL):
                off = base + u * L
                e0 = x0[pl.ds(off, L)] - c0
                e1 = x1[pl.ds(off, L)] - c1
                e2 = x2[pl.ds(off, L)] - c2
                d = e0 * e0 + e1 * e1
                d = d + e2 * e2
                m = d < tau
                ds_.append(d)
                ms_.append(m)
                ns_.append(jnp.max(plsc.all_reduce_population_count(m)))
            total = ns_[0] + ns_[1] + ns_[2] + ns_[3]

            @pl.when(total > 0)
            def _():
                cnt = cnt_s[0]
                for u in range(UNROLL):
                    idxv = lanes + (base + u * L)
                    plsc.store_compressed(cand_v.at[pl.ds(cnt, L)],
                                          ds_[u], mask=ms_[u])
                    plsc.store_compressed(cand_i.at[pl.ds(cnt, L)],
                                          idxv, mask=ms_[u])
                    cnt = cnt + ns_[u]
                cnt_s[0] = cnt

            return 0

        lax.fori_loop(0, N // (L * UNROLL), step, 0)
        rebuild()

        for u in range(M // L):
            iv = cand_i[pl.ds(u * L, L)]
            nb0[pl.ds(u * L, L)] = plsc.load_gather(x0, [iv])
            nb1[pl.ds(u * L, L)] = plsc.load_gather(x1, [iv])
            nb2[pl.ds(u * L, L)] = plsc.load_gather(x2, [iv])
            fdist[pl.ds(u * L, L)] = jnp.full((L,), np.float32(1e10))

        def fps(it, far):
            vstore1(seli, it, far)
            p0 = vload1(nb0, far)
            p1 = vload1(nb1, far)
            p2 = vload1(nb2, far)
            bv = jnp.full((L,), np.float32(-1.0))
            bp = jnp.full((L,), np.int32(M))
            for u in range(M // L):
                e0 = nb0[pl.ds(u * L, L)] - p0
                e1 = nb1[pl.ds(u * L, L)] - p1
                e2 = nb2[pl.ds(u * L, L)] - p2
                dd = e0 * e0 + e1 * e1
                dd = dd + e2 * e2
                nd = jnp.minimum(fdist[pl.ds(u * L, L)], dd)
                fdist[pl.ds(u * L, L)] = nd
                upd = nd > bv
                bv = jnp.where(upd, nd, bv)
                bp = jnp.where(upd, lanes + u * L, bp)
            mx = jnp.max(bv)
            return jnp.min(jnp.where(bv == mx, bp, IMAX))

        lax.fori_loop(0, S, fps, vload1(f0v, cc))

        for u in range(S // L):
            iv = seli[pl.ds(u * L, L)]
            g0 = plsc.load_gather(nb0, [iv])
            g1 = plsc.load_gather(nb1, [iv])
            g2 = plsc.load_gather(nb2, [iv])
            pos = (lanes + u * L) * 3
            plsc.store_scatter(outrow, [pos], g0)
            plsc.store_scatter(outrow, [pos + 1], g1)
            plsc.store_scatter(outrow, [pos + 2], g2)

        pltpu.sync_copy(outrow, out_hbm.at[pl.ds(col * (S * 3), S * 3)])
        return 0

    lax.fori_loop(0, CPW, column, 0)


def _sc_select_fps(xt, ct512, f0):
    mesh = plsc.VectorSubcoreMesh(core_axis_name="c", subcore_axis_name="s")
    fn = pl.kernel(
        _sc_body,
        out_type=jax.ShapeDtypeStruct((COLS * S * 3,), jnp.float32),
        mesh=mesh,
        compiler_params=pltpu.CompilerParams(needs_layout_passes=False),
        scratch_types=[
            pltpu.VMEM((N,), jnp.float32),
            pltpu.VMEM((N,), jnp.float32),
            pltpu.VMEM((N,), jnp.float32),
            pltpu.VMEM((CPW + L,), jnp.float32),
            pltpu.VMEM((CPW + L,), jnp.float32),
            pltpu.VMEM((CPW + L,), jnp.float32),
            pltpu.VMEM((CPW + L,), jnp.int32),
            pltpu.VMEM((CAP,), jnp.float32),
            pltpu.VMEM((CAP,), jnp.int32),
            pltpu.VMEM((M + L,), jnp.float32),
            pltpu.VMEM((M + L,), jnp.int32),
            pltpu.VMEM((M + L,), jnp.float32),
            pltpu.VMEM((M + L,), jnp.float32),
            pltpu.VMEM((M + L,), jnp.float32),
            pltpu.VMEM((M,), jnp.float32),
            pltpu.VMEM((S + L,), jnp.int32),
            pltpu.VMEM((S * 3,), jnp.float32),
            pltpu.SMEM((1,), jnp.int32),
            pltpu.SMEM((1,), jnp.float32),
        ],
    )
    return fn(xt, ct512, f0)


@jax.jit
def kernel(x):
    centers = _kmeans(x)
    ct512 = jnp.transpose(centers, (2, 0, 1)).reshape(3 * COLS)
    xt = jnp.transpose(x, (0, 2, 1)).reshape(B * 3 * N)
    f0 = jax.random.randint(jax.random.key(1), (B, K), 0, M).reshape(COLS)
    out = _sc_select_fps(xt, ct512, f0.astype(jnp.int32))
    new_xyz = out.reshape(B, K * S, 3)
    return new_xyz, centers


# speedup vs baseline: 1.7536x; 1.2047x over previous
import jax
import jax.numpy as jnp
import numpy as np
from jax import lax
from jax.experimental import pallas as pl
from jax.experimental.pallas import tpu as pltpu
from jax.experimental.pallas import tpu_sc as plsc

B = 8
N = 16384
K = 64
M = 64
S = 32
KM_NITER = 10

NC, NS, L = 2, 16, 16
NW = NC * NS
COLS = B * K
CPW = COLS // NW
CAP = 256
REBUILD_AT = CAP - 64
FINF = np.float32(np.inf)
IMAX = np.int32(2**31 - 1)


def _assign_body(x_ref, ct_ref, cl_ref, cnt_ref, sum_ref):
    x = x_ref[0]
    ct = ct_ref[0]
    d0 = x[:, 0:1] - ct[0:1, :]
    d1 = x[:, 1:2] - ct[1:2, :]
    d2c = x[:, 2:3] - ct[2:3, :]
    d = d0 * d0 + d1 * d1
    d = d + d2c * d2c
    cl = jnp.argmin(d, axis=1).astype(jnp.int32)
    cl_ref[0] = cl[:, None]
    oh = (cl[:, None]
          == lax.broadcasted_iota(jnp.int32, (N, K), 1)).astype(jnp.float32)
    cnt_ref[0] = jnp.sum(oh, axis=0, keepdims=True)
    sum_ref[0] = lax.dot_general(x, oh, (((0,), (0,)), ((), ())),
                                 preferred_element_type=jnp.float32,
                                 precision=lax.Precision.HIGHEST)


def _assign(x, ct):
    return pl.pallas_call(
        _assign_body,
        grid=(B,),
        in_specs=[
            pl.BlockSpec((1, N, 3), lambda b: (b, 0, 0)),
            pl.BlockSpec((1, 3, K), lambda b: (b, 0, 0)),
        ],
        out_specs=[
            pl.BlockSpec((1, N, 1), lambda b: (b, 0, 0)),
            pl.BlockSpec((1, 1, K), lambda b: (b, 0, 0)),
            pl.BlockSpec((1, 3, K), lambda b: (b, 0, 0)),
        ],
        out_shape=[
            jax.ShapeDtypeStruct((B, N, 1), jnp.int32),
            jax.ShapeDtypeStruct((B, 1, K), jnp.float32),
            jax.ShapeDtypeStruct((B, 3, K), jnp.float32),
        ],
    )(x, ct)


def _kmeans(x):
    c = x[:, :K, :]
    for i in range(KM_NITER):
        ct = jnp.transpose(c, (0, 2, 1))
        cl3, cnt3, sum3 = _assign(x, ct)
        Ncl = cnt3[:, 0, :]
        if i < KM_NITER - 1:
            cl = cl3[..., 0]
            c = jax.vmap(
                lambda xi, cli: jnp.zeros((K, 3), x.dtype).at[cli].add(xi)
            )(x, cl)
        else:
            c = jnp.transpose(sum3, (0, 2, 1))
        c = c / Ncl[:, :, None]
    return c


def _sc_body(xt_hbm, ct_hbm, f0_hbm, out_hbm,
             x0, x1, x2, cb0, cb1, cb2, f0v,
             cand_v, cand_i, tv, ti, nb0, nb1, nb2, fdist, seli, outrow,
             cnt_s, tau_s):
    wid = lax.axis_index("s") * NC + lax.axis_index("c")
    b = wid // (K // CPW)

    pltpu.sync_copy(xt_hbm.at[pl.ds((b * 3 + 0) * N, N)], x0)
    pltpu.sync_copy(xt_hbm.at[pl.ds((b * 3 + 1) * N, N)], x1)
    pltpu.sync_copy(xt_hbm.at[pl.ds((b * 3 + 2) * N, N)], x2)
    pltpu.sync_copy(ct_hbm.at[pl.ds(0 * COLS + wid * CPW, CPW)],
                    cb0.at[pl.ds(0, CPW)])
    pltpu.sync_copy(ct_hbm.at[pl.ds(1 * COLS + wid * CPW, CPW)],
                    cb1.at[pl.ds(0, CPW)])
    pltpu.sync_copy(ct_hbm.at[pl.ds(2 * COLS + wid * CPW, CPW)],
                    cb2.at[pl.ds(0, CPW)])
    pltpu.sync_copy(f0_hbm.at[pl.ds(wid * CPW, CPW)], f0v.at[pl.ds(0, CPW)])

    lanes = lax.iota(jnp.int32, L)
    lane0 = lanes == 0

    def vload1(ref, idx):
        return ref[pl.ds(idx, L)][0]

    def vstore1(ref, idx, val):
        plsc.store_scatter(ref, [jnp.full((L,), idx, jnp.int32)],
                           jnp.full((L,), val), mask=lane0)

    def rebuild():
        def rnd(r, _):
            def scan1(jj, carry):
                bv, bi = carry
                v = cand_v[pl.ds(jj * L, L)]
                ii = cand_i[pl.ds(jj * L, L)]
                upd = (v < bv) | ((v == bv) & (ii < bi))
                return jnp.where(upd, v, bv), jnp.where(upd, ii, bi)

            bv, bi = lax.fori_loop(0, CAP // L, scan1,
                                   (jnp.full((L,), FINF),
                                    jnp.full((L,), IMAX)))
            mv = jnp.min(bv)
            im = jnp.min(jnp.where(bv == mv, bi, IMAX))

            def clear(jj, _):
                v = cand_v[pl.ds(jj * L, L)]
                ii = cand_i[pl.ds(jj * L, L)]
                hit = (v == mv) & (ii == im)
                cand_v[pl.ds(jj * L, L)] = jnp.where(hit, FINF, v)
                return 0

            lax.fori_loop(0, CAP // L, clear, 0)
            vstore1(tv, r, mv)
            vstore1(ti, r, im)
            tau_s[0] = mv
            return 0

        lax.fori_loop(0, M, rnd, 0)

        def writeback(jj, _):
            cand_v[pl.ds(jj * L, L)] = tv[pl.ds(jj * L, L)]
            cand_i[pl.ds(jj * L, L)] = ti[pl.ds(jj * L, L)]
            return 0

        lax.fori_loop(0, M // L, writeback, 0)

        def fill_inf(jj, _):
            cand_v[pl.ds(M + jj * L, L)] = jnp.full((L,), FINF)
            return 0

        lax.fori_loop(0, (CAP - M) // L, fill_inf, 0)
        cnt_s[0] = M

    def column(cc, _):
        col = wid * CPW + cc
        c0 = vload1(cb0, cc)
        c1 = vload1(cb1, cc)
        c2 = vload1(cb2, cc)

        def fill(jj, _):
            cand_v[pl.ds(jj * L, L)] = jnp.full((L,), FINF)
            return 0

        lax.fori_loop(0, CAP // L, fill, 0)
        cnt_s[0] = 0
        tau_s[0] = FINF

        UNROLL = 4

        def step(j, _):
            @pl.when(cnt_s[0] > REBUILD_AT)
            def _():
                rebuild()

            tau = tau_s[0]
            base = j * (L * UNROLL)
            ds_ = []
            ms_ = []
            ns_ = []
            for u in range(UNROLL):
                off = base + u * L
                e0 = x0[pl.ds(off, L)] - c0
                e1 = x1[pl.ds(off, L)] - c1
                e2 = x2[pl.ds(off, L)] - c2
                d = e0 * e0 + e1 * e1
                d = d + e2 * e2
                m = d < tau
                ds_.append(d)
                ms_.append(m)
                ns_.append(jnp.max(plsc.all_reduce_population_count(m)))
            total = ns_[0] + ns_[1] + ns_[2] + ns_[3]

            @pl.when(total > 0)
            def _():
                cnt = cnt_s[0]
                for u in range(UNROLL):
                    idxv = lanes + (base + u * L)
                    plsc.store_compressed(cand_v.at[pl.ds(cnt, L)],
                                          ds_[u], mask=ms_[u])
                    plsc.store_compressed(cand_i.at[pl.ds(cnt, L)],
                                          idxv, mask=ms_[u])
                    cnt = cnt + ns_[u]
                cnt_s[0] = cnt

            return 0

        lax.fori_loop(0, N // (L * UNROLL), step, 0)
        rebuild()

        for u in range(M // L):
            iv = cand_i[pl.ds(u * L, L)]
            nb0[pl.ds(u * L, L)] = plsc.load_gather(x0, [iv])
            nb1[pl.ds(u * L, L)] = plsc.load_gather(x1, [iv])
            nb2[pl.ds(u * L, L)] = plsc.load_gather(x2, [iv])
            fdist[pl.ds(u * L, L)] = jnp.full((L,), np.float32(1e10))

        def fps(it, far):
            vstore1(seli, it, far)
            p0 = vload1(nb0, far)
            p1 = vload1(nb1, far)
            p2 = vload1(nb2, far)
            bv = jnp.full((L,), np.float32(-1.0))
            bp = jnp.full((L,), np.int32(M))
            for u in range(M // L):
                e0 = nb0[pl.ds(u * L, L)] - p0
                e1 = nb1[pl.ds(u * L, L)] - p1
                e2 = nb2[pl.ds(u * L, L)] - p2
                dd = e0 * e0 + e1 * e1
                dd = dd + e2 * e2
                nd = jnp.minimum(fdist[pl.ds(u * L, L)], dd)
                fdist[pl.ds(u * L, L)] = nd
                upd = nd > bv
                bv = jnp.where(upd, nd, bv)
                bp = jnp.where(upd, lanes + u * L, bp)
            mx = jnp.max(bv)
            return jnp.min(jnp.where(bv == mx, bp, IMAX))

        lax.fori_loop(0, S, fps, vload1(f0v, cc))

        for u in range(S // L):
            iv = seli[pl.ds(u * L, L)]
            g0 = plsc.load_gather(nb0, [iv])
            g1 = plsc.load_gather(nb1, [iv])
            g2 = plsc.load_gather(nb2, [iv])
            pos = (lanes + u * L) * 3
            plsc.store_scatter(outrow, [pos], g0)
            plsc.store_scatter(outrow, [pos + 1], g1)
            plsc.store_scatter(outrow, [pos + 2], g2)

        pltpu.sync_copy(outrow, out_hbm.at[pl.ds(col * (S * 3), S * 3)])
        return 0

    lax.fori_loop(0, CPW, column, 0)


def _sc_select_fps(xt, ct512, f0):
    mesh = plsc.VectorSubcoreMesh(core_axis_name="c", subcore_axis_name="s")
    fn = pl.kernel(
        _sc_body,
        out_type=jax.ShapeDtypeStruct((COLS * S * 3,), jnp.float32),
        mesh=mesh,
        compiler_params=pltpu.CompilerParams(needs_layout_passes=False),
        scratch_types=[
            pltpu.VMEM((N,), jnp.float32),
            pltpu.VMEM((N,), jnp.float32),
            pltpu.VMEM((N,), jnp.float32),
            pltpu.VMEM((CPW + L,), jnp.float32),
            pltpu.VMEM((CPW + L,), jnp.float32),
            pltpu.VMEM((CPW + L,), jnp.float32),
            pltpu.VMEM((CPW + L,), jnp.int32),
            pltpu.VMEM((CAP,), jnp.float32),
            pltpu.VMEM((CAP,), jnp.int32),
            pltpu.VMEM((M + L,), jnp.float32),
            pltpu.VMEM((M + L,), jnp.int32),
            pltpu.VMEM((M + L,), jnp.float32),
            pltpu.VMEM((M + L,), jnp.float32),
            pltpu.VMEM((M + L,), jnp.float32),
            pltpu.VMEM((M,), jnp.float32),
            pltpu.VMEM((S + L,), jnp.int32),
            pltpu.VMEM((S * 3,), jnp.float32),
            pltpu.SMEM((1,), jnp.int32),
            pltpu.SMEM((1,), jnp.float32),
        ],
    )
    return fn(xt, ct512, f0)


@jax.jit
def kernel(x):
    centers = _kmeans(x)
    ct512 = jnp.transpose(centers, (2, 0, 1)).reshape(3 * COLS)
    xt = jnp.transpose(x, (0, 2, 1)).reshape(B * 3 * N)
    f0 = jax.random.randint(jax.random.key(1), (B, K), 0, M).reshape(COLS)
    out = _sc_select_fps(xt, ct512, f0.astype(jnp.int32))
    new_xyz = out.reshape(B, K * S, 3)
    return new_xyz, centers


# speedup vs baseline: 1.7568x; 1.0018x over previous
import jax
import jax.numpy as jnp
import numpy as np
from jax import lax
from jax.experimental import pallas as pl
from jax.experimental.pallas import tpu as pltpu
from jax.experimental.pallas import tpu_sc as plsc

B = 8
N = 16384
K = 64
M = 64
S = 32
KM_NITER = 10

NC, NS, L = 2, 16, 16
NW = NC * NS
COLS = B * K
CPW = COLS // NW
CAP = 256
REBUILD_AT = CAP - 64
FINF = np.float32(np.inf)
IMAX = np.int32(2**31 - 1)


def _assign_body(x_ref, ct_ref, cl_ref, cnt_ref, sum_ref):
    x = x_ref[0]
    ct = ct_ref[0]
    d0 = x[:, 0:1] - ct[0:1, :]
    d1 = x[:, 1:2] - ct[1:2, :]
    d2c = x[:, 2:3] - ct[2:3, :]
    d = d0 * d0 + d1 * d1
    d = d + d2c * d2c
    cl = jnp.argmin(d, axis=1).astype(jnp.int32)
    cl_ref[0] = cl[:, None]
    oh = (cl[:, None]
          == lax.broadcasted_iota(jnp.int32, (N, K), 1)).astype(jnp.float32)
    cnt_ref[0] = jnp.sum(oh, axis=0, keepdims=True)
    sum_ref[0] = lax.dot_general(x, oh, (((0,), (0,)), ((), ())),
                                 preferred_element_type=jnp.float32,
                                 precision=lax.Precision.HIGHEST)


def _assign(x, ct):
    return pl.pallas_call(
        _assign_body,
        grid=(B,),
        in_specs=[
            pl.BlockSpec((1, N, 3), lambda b: (b, 0, 0)),
            pl.BlockSpec((1, 3, K), lambda b: (b, 0, 0)),
        ],
        out_specs=[
            pl.BlockSpec((1, N, 1), lambda b: (b, 0, 0)),
            pl.BlockSpec((1, 1, K), lambda b: (b, 0, 0)),
            pl.BlockSpec((1, 3, K), lambda b: (b, 0, 0)),
        ],
        out_shape=[
            jax.ShapeDtypeStruct((B, N, 1), jnp.int32),
            jax.ShapeDtypeStruct((B, 1, K), jnp.float32),
            jax.ShapeDtypeStruct((B, 3, K), jnp.float32),
        ],
    )(x, ct)


def _kmeans(x):
    c = x[:, :K, :]
    for i in range(KM_NITER):
        ct = jnp.transpose(c, (0, 2, 1))
        cl3, cnt3, sum3 = _assign(x, ct)
        Ncl = cnt3[:, 0, :]
        if i < KM_NITER - 1:
            cl = cl3[..., 0]
            c = jax.vmap(
                lambda xi, cli: jnp.zeros((K, 3), x.dtype).at[cli].add(xi)
            )(x, cl)
        else:
            c = jnp.transpose(sum3, (0, 2, 1))
        c = c / Ncl[:, :, None]
    return c


def _sc_body(xt_hbm, ct_hbm, f0_hbm, out_hbm,
             x0, x1, x2, cb0, cb1, cb2, f0v,
             cand_v, cand_i, tv, ti, nb0, nb1, nb2, fdist, seli, outrow,
             cnt_s, tau_s):
    wid = lax.axis_index("s") * NC + lax.axis_index("c")
    b = wid // (K // CPW)

    pltpu.sync_copy(xt_hbm.at[pl.ds((b * 3 + 0) * N, N)], x0)
    pltpu.sync_copy(xt_hbm.at[pl.ds((b * 3 + 1) * N, N)], x1)
    pltpu.sync_copy(xt_hbm.at[pl.ds((b * 3 + 2) * N, N)], x2)
    pltpu.sync_copy(ct_hbm.at[pl.ds(0 * COLS + wid * CPW, CPW)],
                    cb0.at[pl.ds(0, CPW)])
    pltpu.sync_copy(ct_hbm.at[pl.ds(1 * COLS + wid * CPW, CPW)],
                    cb1.at[pl.ds(0, CPW)])
    pltpu.sync_copy(ct_hbm.at[pl.ds(2 * COLS + wid * CPW, CPW)],
                    cb2.at[pl.ds(0, CPW)])
    pltpu.sync_copy(f0_hbm.at[pl.ds(wid * CPW, CPW)], f0v.at[pl.ds(0, CPW)])

    lanes = lax.iota(jnp.int32, L)
    lane0 = lanes == 0

    def vload1(ref, idx):
        return ref[pl.ds(idx, L)][0]

    def vstore1(ref, idx, val):
        plsc.store_scatter(ref, [jnp.full((L,), idx, jnp.int32)],
                           jnp.full((L,), val), mask=lane0)

    def rebuild():
        nvr = (cnt_s[0] + (L - 1)) // L

        def rnd(r, _):
            def scan1(jj, carry):
                bv, bi = carry
                v = cand_v[pl.ds(jj * L, L)]
                ii = cand_i[pl.ds(jj * L, L)]
                upd = (v < bv) | ((v == bv) & (ii < bi))
                return jnp.where(upd, v, bv), jnp.where(upd, ii, bi)

            bv, bi = lax.fori_loop(0, nvr, scan1,
                                   (jnp.full((L,), FINF),
                                    jnp.full((L,), IMAX)))
            mv = jnp.min(bv)
            im = jnp.min(jnp.where(bv == mv, bi, IMAX))

            def clear(jj, _):
                v = cand_v[pl.ds(jj * L, L)]
                ii = cand_i[pl.ds(jj * L, L)]
                hit = (v == mv) & (ii == im)
                cand_v[pl.ds(jj * L, L)] = jnp.where(hit, FINF, v)
                return 0

            lax.fori_loop(0, nvr, clear, 0)
            vstore1(tv, r, mv)
            vstore1(ti, r, im)
            tau_s[0] = mv
            return 0

        lax.fori_loop(0, M, rnd, 0)

        def writeback(jj, _):
            cand_v[pl.ds(jj * L, L)] = tv[pl.ds(jj * L, L)]
            cand_i[pl.ds(jj * L, L)] = ti[pl.ds(jj * L, L)]
            return 0

        lax.fori_loop(0, M // L, writeback, 0)

        def fill_inf(jj, _):
            cand_v[pl.ds(M + jj * L, L)] = jnp.full((L,), FINF)
            return 0

        lax.fori_loop(0, (CAP - M) // L, fill_inf, 0)
        cnt_s[0] = M

    def column(cc, _):
        col = wid * CPW + cc
        c0 = vload1(cb0, cc)
        c1 = vload1(cb1, cc)
        c2 = vload1(cb2, cc)

        def fill(jj, _):
            cand_v[pl.ds(jj * L, L)] = jnp.full((L,), FINF)
            return 0

        lax.fori_loop(0, CAP // L, fill, 0)
        cnt_s[0] = 0
        tau_s[0] = FINF

        UNROLL = 4

        def step(j, _):
            @pl.when(cnt_s[0] > REBUILD_AT)
            def _():
                rebuild()

            tau = tau_s[0]
            base = j * (L * UNROLL)
            ds_ = []
            ms_ = []
            for u in range(UNROLL):
                off = base + u * L
                e0 = x0[pl.ds(off, L)] - c0
                e1 = x1[pl.ds(off, L)] - c1
                e2 = x2[pl.ds(off, L)] - c2
                d = e0 * e0 + e1 * e1
                d = d + e2 * e2
                ds_.append(d)
                ms_.append(d < tau)
            m_any = (ms_[0] | ms_[1]) | (ms_[2] | ms_[3])
            total = jnp.max(plsc.all_reduce_population_count(m_any))

            @pl.when(total > 0)
            def _():
                cnt = cnt_s[0]
                for u in range(UNROLL):
                    idxv = lanes + (base + u * L)
                    plsc.store_compressed(cand_v.at[pl.ds(cnt, L)],
                                          ds_[u], mask=ms_[u])
                    plsc.store_compressed(cand_i.at[pl.ds(cnt, L)],
                                          idxv, mask=ms_[u])
                    nu = jnp.max(plsc.all_reduce_population_count(ms_[u]))
                    cnt = cnt + nu
                cnt_s[0] = cnt

            return 0

        lax.fori_loop(0, N // (L * UNROLL), step, 0)
        rebuild()

        for u in range(M // L):
            iv = cand_i[pl.ds(u * L, L)]
            nb0[pl.ds(u * L, L)] = plsc.load_gather(x0, [iv])
            nb1[pl.ds(u * L, L)] = plsc.load_gather(x1, [iv])
            nb2[pl.ds(u * L, L)] = plsc.load_gather(x2, [iv])
            fdist[pl.ds(u * L, L)] = jnp.full((L,), np.float32(1e10))

        def fps(it, far):
            vstore1(seli, it, far)
            p0 = vload1(nb0, far)
            p1 = vload1(nb1, far)
            p2 = vload1(nb2, far)
            bv = jnp.full((L,), np.float32(-1.0))
            bp = jnp.full((L,), np.int32(M))
            for u in range(M // L):
                e0 = nb0[pl.ds(u * L, L)] - p0
                e1 = nb1[pl.ds(u * L, L)] - p1
                e2 = nb2[pl.ds(u * L, L)] - p2
                dd = e0 * e0 + e1 * e1
                dd = dd + e2 * e2
                nd = jnp.minimum(fdist[pl.ds(u * L, L)], dd)
                fdist[pl.ds(u * L, L)] = nd
                upd = nd > bv
                bv = jnp.where(upd, nd, bv)
                bp = jnp.where(upd, lanes + u * L, bp)
            mx = jnp.max(bv)
            return jnp.min(jnp.where(bv == mx, bp, IMAX))

        lax.fori_loop(0, S, fps, vload1(f0v, cc))

        for u in range(S // L):
            iv = seli[pl.ds(u * L, L)]
            g0 = plsc.load_gather(nb0, [iv])
            g1 = plsc.load_gather(nb1, [iv])
            g2 = plsc.load_gather(nb2, [iv])
            pos = (lanes + u * L) * 3
            plsc.store_scatter(outrow, [pos], g0)
            plsc.store_scatter(outrow, [pos + 1], g1)
            plsc.store_scatter(outrow, [pos + 2], g2)

        pltpu.sync_copy(outrow, out_hbm.at[pl.ds(col * (S * 3), S * 3)])
        return 0

    lax.fori_loop(0, CPW, column, 0)


def _sc_select_fps(xt, ct512, f0):
    mesh = plsc.VectorSubcoreMesh(core_axis_name="c", subcore_axis_name="s")
    fn = pl.kernel(
        _sc_body,
        out_type=jax.ShapeDtypeStruct((COLS * S * 3,), jnp.float32),
        mesh=mesh,
        compiler_params=pltpu.CompilerParams(needs_layout_passes=False),
        scratch_types=[
            pltpu.VMEM((N,), jnp.float32),
            pltpu.VMEM((N,), jnp.float32),
            pltpu.VMEM((N,), jnp.float32),
            pltpu.VMEM((CPW + L,), jnp.float32),
            pltpu.VMEM((CPW + L,), jnp.float32),
            pltpu.VMEM((CPW + L,), jnp.float32),
            pltpu.VMEM((CPW + L,), jnp.int32),
            pltpu.VMEM((CAP,), jnp.float32),
            pltpu.VMEM((CAP,), jnp.int32),
            pltpu.VMEM((M + L,), jnp.float32),
            pltpu.VMEM((M + L,), jnp.int32),
            pltpu.VMEM((M + L,), jnp.float32),
            pltpu.VMEM((M + L,), jnp.float32),
            pltpu.VMEM((M + L,), jnp.float32),
            pltpu.VMEM((M,), jnp.float32),
            pltpu.VMEM((S + L,), jnp.int32),
            pltpu.VMEM((S * 3,), jnp.float32),
            pltpu.SMEM((1,), jnp.int32),
            pltpu.SMEM((1,), jnp.float32),
        ],
    )
    return fn(xt, ct512, f0)


@jax.jit
def kernel(x):
    centers = _kmeans(x)
    ct512 = jnp.transpose(centers, (2, 0, 1)).reshape(3 * COLS)
    xt = jnp.transpose(x, (0, 2, 1)).reshape(B * 3 * N)
    f0 = jax.random.randint(jax.random.key(1), (B, K), 0, M).reshape(COLS)
    out = _sc_select_fps(xt, ct512, f0.astype(jnp.int32))
    new_xyz = out.reshape(B, K * S, 3)
    return new_xyz, centers


# speedup vs baseline: 1.7813x; 1.0140x over previous
import jax
import jax.numpy as jnp
import numpy as np
from jax import lax
from jax.experimental import pallas as pl
from jax.experimental.pallas import tpu as pltpu
from jax.experimental.pallas import tpu_sc as plsc

B = 8
N = 16384
K = 64
M = 64
S = 32
KM_NITER = 10

NC, NS, L = 2, 16, 16
NW = NC * NS
COLS = B * K
CPW = COLS // NW
CAP = 384
UNROLL = 8
REBUILD_AT = CAP - L * UNROLL
FINF = np.float32(np.inf)
IMAX = np.int32(2**31 - 1)


def _assign_body(x_ref, ct_ref, cl_ref, cnt_ref, sum_ref):
    x = x_ref[0]
    ct = ct_ref[0]
    d0 = x[:, 0:1] - ct[0:1, :]
    d1 = x[:, 1:2] - ct[1:2, :]
    d2c = x[:, 2:3] - ct[2:3, :]
    d = d0 * d0 + d1 * d1
    d = d + d2c * d2c
    cl = jnp.argmin(d, axis=1).astype(jnp.int32)
    cl_ref[0] = cl[:, None]
    oh = (cl[:, None]
          == lax.broadcasted_iota(jnp.int32, (N, K), 1)).astype(jnp.float32)
    cnt_ref[0] = jnp.sum(oh, axis=0, keepdims=True)
    sum_ref[0] = lax.dot_general(x, oh, (((0,), (0,)), ((), ())),
                                 preferred_element_type=jnp.float32,
                                 precision=lax.Precision.HIGHEST)


def _assign(x, ct):
    return pl.pallas_call(
        _assign_body,
        grid=(B,),
        in_specs=[
            pl.BlockSpec((1, N, 3), lambda b: (b, 0, 0)),
            pl.BlockSpec((1, 3, K), lambda b: (b, 0, 0)),
        ],
        out_specs=[
            pl.BlockSpec((1, N, 1), lambda b: (b, 0, 0)),
            pl.BlockSpec((1, 1, K), lambda b: (b, 0, 0)),
            pl.BlockSpec((1, 3, K), lambda b: (b, 0, 0)),
        ],
        out_shape=[
            jax.ShapeDtypeStruct((B, N, 1), jnp.int32),
            jax.ShapeDtypeStruct((B, 1, K), jnp.float32),
            jax.ShapeDtypeStruct((B, 3, K), jnp.float32),
        ],
    )(x, ct)


def _kmeans(x):
    c = x[:, :K, :]
    for i in range(KM_NITER):
        ct = jnp.transpose(c, (0, 2, 1))
        cl3, cnt3, sum3 = _assign(x, ct)
        Ncl = cnt3[:, 0, :]
        if i < KM_NITER - 1:
            cl = cl3[..., 0]
            c = jax.vmap(
                lambda xi, cli: jnp.zeros((K, 3), x.dtype).at[cli].add(xi)
            )(x, cl)
        else:
            c = jnp.transpose(sum3, (0, 2, 1))
        c = c / Ncl[:, :, None]
    return c


def _sc_body(xt_hbm, ct_hbm, f0_hbm, out_hbm,
             x0, x1, x2, cb0, cb1, cb2, f0v,
             cand_v, cand_i, tv, ti, nb0, nb1, nb2, fdist, seli, outrow,
             cnt_s, tau_s):
    wid = lax.axis_index("s") * NC + lax.axis_index("c")
    b = wid // (K // CPW)

    pltpu.sync_copy(xt_hbm.at[pl.ds((b * 3 + 0) * N, N)], x0)
    pltpu.sync_copy(xt_hbm.at[pl.ds((b * 3 + 1) * N, N)], x1)
    pltpu.sync_copy(xt_hbm.at[pl.ds((b * 3 + 2) * N, N)], x2)
    pltpu.sync_copy(ct_hbm.at[pl.ds(0 * COLS + wid * CPW, CPW)],
                    cb0.at[pl.ds(0, CPW)])
    pltpu.sync_copy(ct_hbm.at[pl.ds(1 * COLS + wid * CPW, CPW)],
                    cb1.at[pl.ds(0, CPW)])
    pltpu.sync_copy(ct_hbm.at[pl.ds(2 * COLS + wid * CPW, CPW)],
                    cb2.at[pl.ds(0, CPW)])
    pltpu.sync_copy(f0_hbm.at[pl.ds(wid * CPW, CPW)], f0v.at[pl.ds(0, CPW)])

    lanes = lax.iota(jnp.int32, L)
    lane0 = lanes == 0

    def vload1(ref, idx):
        return ref[pl.ds(idx, L)][0]

    def vstore1(ref, idx, val):
        plsc.store_scatter(ref, [jnp.full((L,), idx, jnp.int32)],
                           jnp.full((L,), val), mask=lane0)

    def rebuild():
        nvr = (cnt_s[0] + (L - 1)) // L

        def rnd(r, _):
            def scan1(jj, carry):
                bv, bi = carry
                v = cand_v[pl.ds(jj * L, L)]
                ii = cand_i[pl.ds(jj * L, L)]
                upd = (v < bv) | ((v == bv) & (ii < bi))
                return jnp.where(upd, v, bv), jnp.where(upd, ii, bi)

            bv, bi = lax.fori_loop(0, nvr, scan1,
                                   (jnp.full((L,), FINF),
                                    jnp.full((L,), IMAX)))
            mv = jnp.min(bv)
            im = jnp.min(jnp.where(bv == mv, bi, IMAX))

            def clear(jj, _):
                v = cand_v[pl.ds(jj * L, L)]
                ii = cand_i[pl.ds(jj * L, L)]
                hit = (v == mv) & (ii == im)
                cand_v[pl.ds(jj * L, L)] = jnp.where(hit, FINF, v)
                return 0

            lax.fori_loop(0, nvr, clear, 0)
            vstore1(tv, r, mv)
            vstore1(ti, r, im)
            tau_s[0] = mv
            return 0

        lax.fori_loop(0, M, rnd, 0)

        def writeback(jj, _):
            cand_v[pl.ds(jj * L, L)] = tv[pl.ds(jj * L, L)]
            cand_i[pl.ds(jj * L, L)] = ti[pl.ds(jj * L, L)]
            return 0

        lax.fori_loop(0, M // L, writeback, 0)

        def fill_inf(jj, _):
            cand_v[pl.ds(M + jj * L, L)] = jnp.full((L,), FINF)
            return 0

        lax.fori_loop(0, (CAP - M) // L, fill_inf, 0)
        cnt_s[0] = M

    def column(cc, _):
        col = wid * CPW + cc
        c0 = vload1(cb0, cc)
        c1 = vload1(cb1, cc)
        c2 = vload1(cb2, cc)

        def fill(jj, _):
            cand_v[pl.ds(jj * L, L)] = jnp.full((L,), FINF)
            return 0

        lax.fori_loop(0, CAP // L, fill, 0)
        cnt_s[0] = 0
        tau_s[0] = FINF

        def step(j, _):
            @pl.when(cnt_s[0] > REBUILD_AT)
            def _():
                rebuild()

            tau = tau_s[0]
            base = j * (L * UNROLL)
            ds_ = []
            ms_ = []
            for u in range(UNROLL):
                off = base + u * L
                e0 = x0[pl.ds(off, L)] - c0
                e1 = x1[pl.ds(off, L)] - c1
                e2 = x2[pl.ds(off, L)] - c2
                d = e0 * e0 + e1 * e1
                d = d + e2 * e2
                ds_.append(d)
                ms_.append(d < tau)
            m_any = ms_[0]
            for u in range(1, UNROLL):
                m_any = m_any | ms_[u]
            total = jnp.max(plsc.all_reduce_population_count(m_any))

            @pl.when(total > 0)
            def _():
                cnt = cnt_s[0]
                for u in range(UNROLL):
                    idxv = lanes + (base + u * L)
                    plsc.store_compressed(cand_v.at[pl.ds(cnt, L)],
                                          ds_[u], mask=ms_[u])
                    plsc.store_compressed(cand_i.at[pl.ds(cnt, L)],
                                          idxv, mask=ms_[u])
                    nu = jnp.max(plsc.all_reduce_population_count(ms_[u]))
                    cnt = cnt + nu
                cnt_s[0] = cnt

            return 0

        lax.fori_loop(0, N // (L * UNROLL), step, 0)
        rebuild()

        for u in range(M // L):
            iv = cand_i[pl.ds(u * L, L)]
            nb0[pl.ds(u * L, L)] = plsc.load_gather(x0, [iv])
            nb1[pl.ds(u * L, L)] = plsc.load_gather(x1, [iv])
            nb2[pl.ds(u * L, L)] = plsc.load_gather(x2, [iv])
            fdist[pl.ds(u * L, L)] = jnp.full((L,), np.float32(1e10))

        def fps(it, far):
            vstore1(seli, it, far)
            p0 = vload1(nb0, far)
            p1 = vload1(nb1, far)
            p2 = vload1(nb2, far)
            bv = jnp.full((L,), np.float32(-1.0))
            bp = jnp.full((L,), np.int32(M))
            for u in range(M // L):
                e0 = nb0[pl.ds(u * L, L)] - p0
                e1 = nb1[pl.ds(u * L, L)] - p1
                e2 = nb2[pl.ds(u * L, L)] - p2
                dd = e0 * e0 + e1 * e1
                dd = dd + e2 * e2
                nd = jnp.minimum(fdist[pl.ds(u * L, L)], dd)
                fdist[pl.ds(u * L, L)] = nd
                upd = nd > bv
                bv = jnp.where(upd, nd, bv)
                bp = jnp.where(upd, lanes + u * L, bp)
            mx = jnp.max(bv)
            return jnp.min(jnp.where(bv == mx, bp, IMAX))

        lax.fori_loop(0, S, fps, vload1(f0v, cc))

        for u in range(S // L):
            iv = seli[pl.ds(u * L, L)]
            g0 = plsc.load_gather(nb0, [iv])
            g1 = plsc.load_gather(nb1, [iv])
            g2 = plsc.load_gather(nb2, [iv])
            pos = (lanes + u * L) * 3
            plsc.store_scatter(outrow, [pos], g0)
            plsc.store_scatter(outrow, [pos + 1], g1)
            plsc.store_scatter(outrow, [pos + 2], g2)

        pltpu.sync_copy(outrow, out_hbm.at[pl.ds(col * (S * 3), S * 3)])
        return 0

    lax.fori_loop(0, CPW, column, 0)


def _sc_select_fps(xt, ct512, f0):
    mesh = plsc.VectorSubcoreMesh(core_axis_name="c", subcore_axis_name="s")
    fn = pl.kernel(
        _sc_body,
        out_type=jax.ShapeDtypeStruct((COLS * S * 3,), jnp.float32),
        mesh=mesh,
        compiler_params=pltpu.CompilerParams(needs_layout_passes=False),
        scratch_types=[
            pltpu.VMEM((N,), jnp.float32),
            pltpu.VMEM((N,), jnp.float32),
            pltpu.VMEM((N,), jnp.float32),
            pltpu.VMEM((CPW + L,), jnp.float32),
            pltpu.VMEM((CPW + L,), jnp.float32),
            pltpu.VMEM((CPW + L,), jnp.float32),
            pltpu.VMEM((CPW + L,), jnp.int32),
            pltpu.VMEM((CAP,), jnp.float32),
            pltpu.VMEM((CAP,), jnp.int32),
            pltpu.VMEM((M + L,), jnp.float32),
            pltpu.VMEM((M + L,), jnp.int32),
            pltpu.VMEM((M + L,), jnp.float32),
            pltpu.VMEM((M + L,), jnp.float32),
            pltpu.VMEM((M + L,), jnp.float32),
            pltpu.VMEM((M,), jnp.float32),
            pltpu.VMEM((S + L,), jnp.int32),
            pltpu.VMEM((S * 3,), jnp.float32),
            pltpu.SMEM((1,), jnp.int32),
            pltpu.SMEM((1,), jnp.float32),
        ],
    )
    return fn(xt, ct512, f0)


@jax.jit
def kernel(x):
    centers = _kmeans(x)
    ct512 = jnp.transpose(centers, (2, 0, 1)).reshape(3 * COLS)
    xt = jnp.transpose(x, (0, 2, 1)).reshape(B * 3 * N)
    f0 = jax.random.randint(jax.random.key(1), (B, K), 0, M).reshape(COLS)
    out = _sc_select_fps(xt, ct512, f0.astype(jnp.int32))
    new_xyz = out.reshape(B, K * S, 3)
    return new_xyz, centers
